# baseline (speedup 1.0000x reference)
import functools

import jax
import jax.numpy as jnp
from jax import lax
from jax.experimental import pallas as pl
from jax.experimental.pallas import tpu as pltpu
from jax.experimental.pallas import tpu_sc as plsc

E = 64
D = 768
DFF = 1536
DE = DFF // E
N = 8192
P = 2 * N
TM = 128
NB = P // TM + E - 1
NP = (NB + 1) * TM
NW = 32
TPW = N // NW
CH = 64
NCH = TPW // CH
GW = 128
D2 = D // 2
C5 = 32
NC5 = TPW // C5

TB1 = 1024


def _k1_body(x_ref, cen_ref, wr_ref, idx_ref, gate_ref):
    x = x_ref[...]
    cen = cen_ref[...]
    cn = cen / (jnp.sqrt(jnp.sum(cen * cen, axis=1, keepdims=True)) + 1e-12)
    xn = x / (jnp.sqrt(jnp.sum(x * x, axis=1, keepdims=True)) + 1e-12)
    nt = (((1,), (1,)), ((), ()))
    logits = (lax.dot_general(xn, cn, nt, preferred_element_type=jnp.float32)
              + lax.dot_general(x, wr_ref[...], nt, preferred_element_type=jnp.float32))
    ie = lax.broadcasted_iota(jnp.int32, (TB1, E), 1)
    m1 = jnp.max(logits, axis=1, keepdims=True)
    i1 = jnp.min(jnp.where(logits >= m1, ie, E), axis=1, keepdims=True)
    l2 = jnp.where(ie == i1, -jnp.inf, logits)
    m2 = jnp.max(l2, axis=1, keepdims=True)
    i2 = jnp.min(jnp.where(l2 >= m2, ie, E), axis=1, keepdims=True)
    d = jnp.exp(m2 - m1)
    g1 = 1.0 / (1.0 + d)
    g2 = d * g1
    idx_ref[...] = jnp.concatenate([i1, i2], axis=1)
    gate_ref[...] = jnp.concatenate([g1, g2], axis=1)


def _route(xf, centroids, Wr):
    full = lambda i: (0, 0)
    return pl.pallas_call(
        _k1_body,
        grid=(N // TB1,),
        in_specs=[
            pl.BlockSpec((TB1, D), lambda i: (i, 0)),
            pl.BlockSpec((E, D), full),
            pl.BlockSpec((E, D), full),
        ],
        out_specs=[
            pl.BlockSpec((TB1, 2), lambda i: (i, 0)),
            pl.BlockSpec((TB1, 2), lambda i: (i, 0)),
        ],
        out_shape=[
            jax.ShapeDtypeStruct((N, 2), jnp.int32),
            jax.ShapeDtypeStruct((N, 2), jnp.float32),
        ],
    )(xf, centroids, Wr)


KC = 512
NKC = P // KC
NBPAD = 256


def _k2_body(e_ref, pos_ref, bexp_ref):
    iexp = lax.broadcasted_iota(jnp.int32, (E, 1), 0).astype(jnp.float32)
    triU = (lax.broadcasted_iota(jnp.int32, (KC, KC), 0)
            <= lax.broadcasted_iota(jnp.int32, (KC, KC), 1)
            ).astype(jnp.bfloat16)
    ntf = (((1,), (0,)), ((), ()))

    def onehot_t(c):
        er = e_ref[pl.ds(c, 1), :].astype(jnp.float32)
        return (iexp == er).astype(jnp.float32)

    def count_step(c, carry):
        return carry + jnp.sum(onehot_t(c), axis=1, keepdims=True)

    counts = lax.fori_loop(0, NKC, count_step,
                           jnp.zeros((E, 1), jnp.float32))
    padded = jnp.floor((counts + (TM - 1)) * (1.0 / TM)) * TM
    Lstrict = (lax.broadcasted_iota(jnp.int32, (E, E), 1)
               < lax.broadcasted_iota(jnp.int32, (E, E), 0)).astype(jnp.float32)
    off = lax.dot_general(Lstrict, padded, ntf,
                          preferred_element_type=jnp.float32)

    def pos_step(c, carry):
        oh = onehot_t(c)
        ohb = oh.astype(jnp.bfloat16)
        rank = lax.dot_general(ohb, triU, ntf,
                               preferred_element_type=jnp.float32)
        base = off + carry
        posc = jnp.sum(oh * (rank + base), axis=0, keepdims=True) - 1.0
        pos_ref[pl.ds(c, 1), :] = posc.astype(jnp.int32)
        return carry + jnp.sum(oh, axis=1, keepdims=True)

    lax.fori_loop(0, NKC, pos_step, jnp.zeros((E, 1), jnp.float32))

    r = lax.broadcasted_iota(jnp.int32, (1, NBPAD), 1).astype(jnp.float32) * TM
    be = jnp.sum((off <= r).astype(jnp.float32), axis=0, keepdims=True) - 1.0
    total = jnp.sum(padded)
    bexp_ref[...] = jnp.where(r < total, be, -1.0).astype(jnp.int32)


def _dispatch(e2d):
    return pl.pallas_call(
        _k2_body,
        in_specs=[pl.BlockSpec((NKC, KC), lambda: (0, 0))],
        out_specs=[
            pl.BlockSpec((NKC, KC), lambda: (0, 0)),
            pl.BlockSpec((1, NBPAD), lambda: (0, 0)),
        ],
        out_shape=[
            jax.ShapeDtypeStruct((NKC, KC), jnp.int32),
            jax.ShapeDtypeStruct((1, NBPAD), jnp.int32),
        ],
    )(e2d)


def _k3_body(xbf, pos0, pos1, g0r, g1r, xs, gs,
             p0_v, p1_v, bx0, bx1, g0a, g0b, g1a, g1b, semG, semS):
    w = lax.axis_index("s") * 2 + lax.axis_index("c")
    pltpu.sync_copy(pos0.at[w], p0_v)
    pltpu.sync_copy(pos1.at[w], p1_v)
    sets = ((bx0, g0a, g1a), (bx1, g0b, g1b))

    def issue_gather(c):
        bx, g0, g1 = sets[c % 2]
        return (pltpu.async_copy(xbf.at[pl.ds(w * TPW + c * CH, CH)], bx, semG),
                pltpu.async_copy(g0r.at[w, c], g0, semG),
                pltpu.async_copy(g1r.at[w, c], g1, semG))

    def issue_scatter(c):
        bx, g0, g1 = sets[c % 2]
        return (pltpu.async_copy(bx, xs.at[p0_v.at[c]], semS),
                pltpu.async_copy(bx, xs.at[p1_v.at[c]], semS),
                pltpu.async_copy(g0, gs.at[p0_v.at[c]], semS),
                pltpu.async_copy(g1, gs.at[p1_v.at[c]], semS))

    pend = [None, None]
    g = issue_gather(0)
    for c in range(NCH):
        for d in g:
            d.wait()
        if c + 1 < NCH:
            nxt = (c + 1) % 2
            if pend[nxt] is not None:
                for d in pend[nxt]:
                    d.wait()
                pend[nxt] = None
            g = issue_gather(c + 1)
        pend[c % 2] = issue_scatter(c)
    for p in pend:
        if p is not None:
            for d in p:
                d.wait()


def _scatter_tokens(xbf, pos0, pos1, g0r, g1r):
    mesh = plsc.VectorSubcoreMesh(core_axis_name="c", subcore_axis_name="s")
    kfn = pl.kernel(
        _k3_body,
        mesh=mesh,
        out_type=[
            jax.ShapeDtypeStruct((NP, D2), jnp.float32),
            jax.ShapeDtypeStruct((NP, GW), jnp.float32),
        ],
        scratch_types=[
            pltpu.VMEM((NCH, CH), jnp.int32),
            pltpu.VMEM((NCH, CH), jnp.int32),
            pltpu.VMEM((CH, D2), jnp.float32),
            pltpu.VMEM((CH, D2), jnp.float32),
            pltpu.VMEM((CH, GW), jnp.float32),
            pltpu.VMEM((CH, GW), jnp.float32),
            pltpu.VMEM((CH, GW), jnp.float32),
            pltpu.VMEM((CH, GW), jnp.float32),
            pltpu.SemaphoreType.DMA,
            pltpu.SemaphoreType.DMA,
        ],
    )
    return kfn(xbf, pos0, pos1, g0r, g1r)


SB = 4
TMS = SB * TM
NBS = NP // TMS
NP2 = (NBS + 1) * TMS


def _k4_body(bexp_ref, xs_ref, gs_ref, *rest):
    (w1a, w1b, w1c, w1d, b1a, b1b, b1c, b1d,
     w2a, w2b, w2c, w2d, b2a, b2b, b2c, b2d, ys_ref) = rest
    w1r = (w1a, w1b, w1c, w1d)
    b1r = (b1a, b1b, b1c, b1d)
    w2r = (w2a, w2b, w2c, w2d)
    b2r = (b2a, b2b, b2c, b2d)
    j = pl.program_id(0)
    nt = (((1,), (1,)), ((), ()))
    for b in range(SB):
        @pl.when(bexp_ref[SB * j + b] >= 0)
        def _(b=b):
            rs = pl.ds(b * TM, TM)
            x2 = xs_ref[rs, :]
            xi = lax.bitcast_convert_type(x2, jnp.int32)
            xlo = lax.bitcast_convert_type(xi << 16, jnp.float32)
            xhi = lax.bitcast_convert_type(
                xi & jnp.int32(-65536), jnp.float32)
            xcat = jnp.concatenate([xlo, xhi], axis=1)
            g = gs_ref[rs, 0:1]
            h = lax.dot_general(xcat, w1r[b][0], nt,
                                preferred_element_type=jnp.float32)
            h = h + b1r[b][0]
            h = 0.5 * h * (1.0 + lax.erf(h * 0.7071067811865476))
            hg = h * g
            y = lax.dot_general(hg, w2r[b][0], nt,
                                preferred_element_type=jnp.float32)
            ys_ref[rs, :] = y + g * b2r[b][0]


def _expert_ffn(bexp, xs, gs, w1, b1, w2, b2):
    def _e(b):
        return lambda i, be: (jnp.maximum(be[SB * i + b], 0), 0, 0)

    def _data(i, be):
        return (jnp.where(be[SB * i] < 0, 0, i), 0)

    grid_spec = pltpu.PrefetchScalarGridSpec(
        num_scalar_prefetch=1,
        grid=(NBS,),
        in_specs=[
            pl.BlockSpec((TMS, D2), _data),
            pl.BlockSpec((TMS, GW), _data),
        ]
        + [pl.BlockSpec((1, DE, D), _e(b)) for b in range(SB)]
        + [pl.BlockSpec((1, 1, DE), _e(b)) for b in range(SB)]
        + [pl.BlockSpec((1, D, DE), _e(b)) for b in range(SB)]
        + [pl.BlockSpec((1, 1, D), _e(b)) for b in range(SB)],
        out_specs=pl.BlockSpec(
            (TMS, D), lambda i, be: (jnp.where(be[SB * i] < 0, NBS, i), 0)),
    )
    b1r = b1.reshape(E, 1, DE)
    b2r = b2.reshape(E, 1, D)
    w1p = jnp.concatenate([w1[:, :, 0::2], w1[:, :, 1::2]], axis=2)
    return pl.pallas_call(
        _k4_body,
        grid_spec=grid_spec,
        out_shape=jax.ShapeDtypeStruct((NP2, D), jnp.float32),
    )(bexp, xs, gs, w1p, w1p, w1p, w1p, b1r, b1r, b1r, b1r,
      w2, w2, w2, w2, b2r, b2r, b2r, b2r)


def _k5_body(ys, pos0, pos1, out, p0_v, p1_v, a0, b0, a1, b1, semG, semS):
    w = lax.axis_index("s") * 2 + lax.axis_index("c")
    pltpu.sync_copy(pos0.at[w], p0_v)
    pltpu.sync_copy(pos1.at[w], p1_v)
    sets = ((a0, b0), (a1, b1))

    def issue_gather(c):
        bA, bB = sets[c % 2]
        return (pltpu.async_copy(ys.at[p0_v.at[c]], bA, semG),
                pltpu.async_copy(ys.at[p1_v.at[c]], bB, semG))

    pend = [None, None]
    g = issue_gather(0)
    for c in range(NC5):
        for d in g:
            d.wait()
        bA, bB = sets[c % 2]
        if c + 1 < NC5:
            nxt = (c + 1) % 2
            if pend[nxt] is not None:
                pend[nxt].wait()
                pend[nxt] = None
            g = issue_gather(c + 1)

        def row(r, carry):
            for q in range(D // 16):
                s = pl.ds(q * 16, 16)
                bA[r, s] = bA[r, s] + bB[r, s]
            return carry

        lax.fori_loop(0, C5, row, 0)
        pend[c % 2] = pltpu.async_copy(
            bA, out.at[pl.ds(w * TPW + c * C5, C5)], semS)
    for p in pend:
        if p is not None:
            p.wait()


def _combine(ys, pos0, pos1):
    mesh = plsc.VectorSubcoreMesh(core_axis_name="c", subcore_axis_name="s")
    kfn = pl.kernel(
        _k5_body,
        mesh=mesh,
        out_type=jax.ShapeDtypeStruct((N, D), jnp.float32),
        scratch_types=[
            pltpu.VMEM((NC5, C5), jnp.int32),
            pltpu.VMEM((NC5, C5), jnp.int32),
            pltpu.VMEM((C5, D), jnp.float32),
            pltpu.VMEM((C5, D), jnp.float32),
            pltpu.VMEM((C5, D), jnp.float32),
            pltpu.VMEM((C5, D), jnp.float32),
            pltpu.SemaphoreType.DMA,
            pltpu.SemaphoreType.DMA,
        ],
    )
    return kfn(ys, pos0, pos1)


def kernel(x, centroids, Wr, w1, b1, w2, b2):
    B, S, _ = x.shape
    xf = x.reshape(N, D)

    idx, gate = _route(xf, centroids, Wr)

    e2d = jnp.concatenate([idx[:, 0], idx[:, 1]]).reshape(NKC, KC)
    posm, bexp2 = _dispatch(e2d)

    pos_all = posm.reshape(2, N)
    pos0 = pos_all[0].reshape(NW, NCH, CH)
    pos1 = pos_all[1].reshape(NW, NCH, CH)
    g0r = jnp.broadcast_to(gate[:, 0:1], (N, GW)).reshape(NW, NCH, CH, GW)
    g1r = jnp.broadcast_to(gate[:, 1:2], (N, GW)).reshape(NW, NCH, CH, GW)

    x2 = lax.bitcast_convert_type(
        xf.astype(jnp.bfloat16).reshape(N, D2, 2), jnp.float32)
    xs, gs = _scatter_tokens(x2, pos0, pos1, g0r, g1r)
    ys = _expert_ffn(bexp2.reshape(NBPAD), xs, gs, w1, b1, w2, b2)
    pos0b = pos_all[0].reshape(NW, NC5, C5)
    pos1b = pos_all[1].reshape(NW, NC5, C5)
    out = _combine(ys, pos0b, pos1b)
    return out.reshape(B, S, D)


# speedup vs baseline: 7.1385x; 7.1385x over previous
import functools

import jax
import jax.numpy as jnp
from jax import lax
from jax.experimental import pallas as pl

E = 64
TOPK = 2
D = 768
DFF = 1536
DE = DFF // E
TB = 1024


def _body(x_ref, cen_ref, wr_ref, w1_ref, b1_ref, w2_ref, b2_ref, o_ref):
    x = x_ref[...]
    cen = cen_ref[...]
    wr = wr_ref[...]

    cn = cen / (jnp.sqrt(jnp.sum(cen * cen, axis=1, keepdims=True)) + 1e-12)
    xn = x / (jnp.sqrt(jnp.sum(x * x, axis=1, keepdims=True)) + 1e-12)
    nt = (((1,), (1,)), ((), ()))
    cos = lax.dot_general(xn, cn, nt, preferred_element_type=jnp.float32)
    route = lax.dot_general(x, wr, nt, preferred_element_type=jnp.float32)
    logits = cos + route

    ie = lax.broadcasted_iota(jnp.int32, (TB, E), 1)
    m1 = jnp.max(logits, axis=1, keepdims=True)
    i1 = jnp.min(jnp.where(logits >= m1, ie, E), axis=1, keepdims=True)
    l2 = jnp.where(ie == i1, -jnp.inf, logits)
    m2 = jnp.max(l2, axis=1, keepdims=True)
    i2 = jnp.min(jnp.where(l2 >= m2, ie, E), axis=1, keepdims=True)
    d = jnp.exp(m2 - m1)
    g1 = 1.0 / (1.0 + d)
    g2 = d * g1
    gate = jnp.where(ie == i1, g1, 0.0) + jnp.where(ie == i2, g2, 0.0)

    h = lax.dot_general(x, w1_ref[...], nt, preferred_element_type=jnp.float32)
    h = h + b1_ref[...]
    h = 0.5 * h * (1.0 + lax.erf(h * 0.7071067811865476))

    ce = lax.broadcasted_iota(jnp.int32, (E, DFF), 1)
    re = lax.broadcasted_iota(jnp.int32, (E, DFF), 0) * DE
    R = ((ce >= re) & (ce < re + DE)).astype(jnp.float32)
    gw = lax.dot_general(gate, R, (((1,), (0,)), ((), ())),
                         preferred_element_type=jnp.float32)

    out = lax.dot_general(h * gw, w2_ref[...], (((1,), (0,)), ((), ())),
                          preferred_element_type=jnp.float32)
    out = out + lax.dot_general(gate, b2_ref[...], (((1,), (0,)), ((), ())),
                                preferred_element_type=jnp.float32)
    o_ref[...] = out


def kernel(x, centroids, Wr, w1, b1, w2, b2):
    B, S, _ = x.shape
    N = B * S
    xf = x.reshape(N, D)
    w1f = w1.reshape(DFF, D)
    w2f = jnp.transpose(w2, (0, 2, 1)).reshape(DFF, D)
    b1f = b1.reshape(1, DFF)

    grid = N // TB
    full = lambda *_: (0, 0)
    out = pl.pallas_call(
        _body,
        grid=(grid,),
        in_specs=[
            pl.BlockSpec((TB, D), lambda i: (i, 0)),
            pl.BlockSpec((E, D), full),
            pl.BlockSpec((E, D), full),
            pl.BlockSpec((DFF, D), full),
            pl.BlockSpec((1, DFF), full),
            pl.BlockSpec((DFF, D), full),
            pl.BlockSpec((E, D), full),
        ],
        out_specs=pl.BlockSpec((TB, D), lambda i: (i, 0)),
        out_shape=jax.ShapeDtypeStruct((N, D), jnp.float32),
    )(xf, centroids, Wr, w1f, b1f, w2f, b2)
    return out.reshape(B, S, D)


# speedup vs baseline: 7.1415x; 1.0004x over previous
import functools

import jax
import jax.numpy as jnp
import numpy as np
from jax import lax
from jax.experimental import pallas as pl

E = 64
TOPK = 2
D = 768
DFF = 1536
DE = DFF // E
TB = 1024

_RMAT = jnp.asarray(
    (np.arange(DFF)[None, :] // DE) == np.arange(E)[:, None],
    dtype=jnp.float32)


def _body(x_ref, cen_ref, wr_ref, w1_ref, b1_ref, w2_ref, b2_ref, r_ref, o_ref):
    x = x_ref[...]
    cen = cen_ref[...]
    wr = wr_ref[...]

    cn = cen / (jnp.sqrt(jnp.sum(cen * cen, axis=1, keepdims=True)) + 1e-12)
    xn = x / (jnp.sqrt(jnp.sum(x * x, axis=1, keepdims=True)) + 1e-12)
    nt = (((1,), (1,)), ((), ()))
    cos = lax.dot_general(xn, cn, nt, preferred_element_type=jnp.float32)
    route = lax.dot_general(x, wr, nt, preferred_element_type=jnp.float32)
    logits = cos + route

    ie = lax.broadcasted_iota(jnp.int32, (TB, E), 1)
    m1 = jnp.max(logits, axis=1, keepdims=True)
    i1 = jnp.min(jnp.where(logits >= m1, ie, E), axis=1, keepdims=True)
    l2 = jnp.where(ie == i1, -jnp.inf, logits)
    m2 = jnp.max(l2, axis=1, keepdims=True)
    i2 = jnp.min(jnp.where(l2 >= m2, ie, E), axis=1, keepdims=True)
    d = jnp.exp(m2 - m1)
    g1 = 1.0 / (1.0 + d)
    g2 = d * g1
    gate = jnp.where(ie == i1, g1, 0.0) + jnp.where(ie == i2, g2, 0.0)

    h = lax.dot_general(x, w1_ref[...], nt, preferred_element_type=jnp.float32)
    h = h + b1_ref[...]
    h = 0.5 * h * (1.0 + lax.erf(h * 0.7071067811865476))

    gw = lax.dot_general(gate, r_ref[...], (((1,), (0,)), ((), ())),
                         preferred_element_type=jnp.float32)

    out = lax.dot_general(h * gw, w2_ref[...], (((1,), (0,)), ((), ())),
                          preferred_element_type=jnp.float32)
    out = out + lax.dot_general(gate, b2_ref[...], (((1,), (0,)), ((), ())),
                                preferred_element_type=jnp.float32)
    o_ref[...] = out


def kernel(x, centroids, Wr, w1, b1, w2, b2):
    B, S, _ = x.shape
    N = B * S
    xf = x.reshape(N, D)
    w1f = w1.reshape(DFF, D)
    w2f = jnp.transpose(w2, (0, 2, 1)).reshape(DFF, D)
    b1f = b1.reshape(1, DFF)

    grid = N // TB
    full = lambda *_: (0, 0)
    out = pl.pallas_call(
        _body,
        grid=(grid,),
        in_specs=[
            pl.BlockSpec((TB, D), lambda i: (i, 0)),
            pl.BlockSpec((E, D), full),
            pl.BlockSpec((E, D), full),
            pl.BlockSpec((DFF, D), full),
            pl.BlockSpec((1, DFF), full),
            pl.BlockSpec((DFF, D), full),
            pl.BlockSpec((E, D), full),
            pl.BlockSpec((E, DFF), full),
        ],
        out_specs=pl.BlockSpec((TB, D), lambda i: (i, 0)),
        out_shape=jax.ShapeDtypeStruct((N, D), jnp.float32),
    )(xf, centroids, Wr, w1f, b1f, w2f, b2, _RMAT)
    return out.reshape(B, S, D)


# speedup vs baseline: 7.1424x; 1.0001x over previous
import jax
import jax.numpy as jnp
import numpy as np
from jax import lax
from jax.experimental import pallas as pl

E = 64
TOPK = 2
D = 768
DFF = 1536
DE = DFF // E
TB = 1024

_RMAT = jnp.asarray(
    (np.arange(DFF)[None, :] // DE) == np.arange(E)[:, None],
    dtype=jnp.float32)


def _body(x_ref, cen_ref, wr_ref, w1_ref, b1_ref, w2_ref, b2_ref, r_ref, o_ref):
    x = x_ref[...]
    cen = cen_ref[...]
    wr = wr_ref[...]

    cn = cen / (jnp.sqrt(jnp.sum(cen * cen, axis=1, keepdims=True)) + 1e-12)
    xn = x / (jnp.sqrt(jnp.sum(x * x, axis=1, keepdims=True)) + 1e-12)
    nt = (((1,), (1,)), ((), ()))
    cos = lax.dot_general(xn, cn, nt, preferred_element_type=jnp.float32)
    route = lax.dot_general(x, wr, nt, preferred_element_type=jnp.float32)
    logits = cos + route

    ie = lax.broadcasted_iota(jnp.int32, (TB, E), 1)
    m1 = jnp.max(logits, axis=1, keepdims=True)
    i1 = jnp.min(jnp.where(logits >= m1, ie, E), axis=1, keepdims=True)
    l2 = jnp.where(ie == i1, -jnp.inf, logits)
    m2 = jnp.max(l2, axis=1, keepdims=True)
    i2 = jnp.min(jnp.where(l2 >= m2, ie, E), axis=1, keepdims=True)
    d = jnp.exp(m2 - m1)
    g1 = 1.0 / (1.0 + d)
    g2 = d * g1
    gate = jnp.where(ie == i1, g1, 0.0) + jnp.where(ie == i2, g2, 0.0)

    h = lax.dot_general(x, w1_ref[...], nt, preferred_element_type=jnp.float32)
    h = h + b1_ref[...]
    h = 0.5 * h * (1.0 + lax.erf(h * 0.7071067811865476))

    gw = lax.dot_general(gate, r_ref[...], (((1,), (0,)), ((), ())),
                         preferred_element_type=jnp.float32)

    out = lax.dot_general(h * gw, w2_ref[...], (((1,), (0,)), ((), ())),
                          preferred_element_type=jnp.float32)
    out = out + lax.dot_general(gate, b2_ref[...], (((1,), (0,)), ((), ())),
                                preferred_element_type=jnp.float32)
    o_ref[...] = out


def kernel(x, centroids, Wr, w1, b1, w2, b2):
    B, S, _ = x.shape
    N = B * S
    xf = x.reshape(N, D)
    w1f = w1.reshape(DFF, D)
    w2f = jnp.transpose(w2, (0, 2, 1)).reshape(DFF, D)
    b1f = b1.reshape(1, DFF)

    grid = N // TB
    full = lambda *_: (0, 0)
    out = pl.pallas_call(
        _body,
        grid=(grid,),
        in_specs=[
            pl.BlockSpec((TB, D), lambda i: (i, 0)),
            pl.BlockSpec((E, D), full),
            pl.BlockSpec((E, D), full),
            pl.BlockSpec((DFF, D), full),
            pl.BlockSpec((1, DFF), full),
            pl.BlockSpec((DFF, D), full),
            pl.BlockSpec((E, D), full),
            pl.BlockSpec((E, DFF), full),
        ],
        out_specs=pl.BlockSpec((TB, D), lambda i: (i, 0)),
        out_shape=jax.ShapeDtypeStruct((N, D), jnp.float32),
    )(xf, centroids, Wr, w1f, b1f, w2f, b2, _RMAT)
    return out.reshape(B, S, D)
